# baseline (speedup 1.0000x reference)
import jax
import jax.numpy as jnp
from jax.experimental import pallas as pl
from jax.experimental.pallas import tpu as pltpu

B, FM, H, O, D = 2, 8, 4, 8, 2048
HALF = D // 2
NF = B * FM
NE = NF * H * O
NH = NF * H
G3 = 3 * D
QW = 256
QG = 256
NGH = G3 // QG
QS = 512
NS = G3 // QS
QB = 512
NQ = G3 // QB
MSG_STEPS = 14
BF = jnp.bfloat16
F32 = jnp.float32


def _bdot(x, w):
    return jax.lax.dot_general(
        x.astype(BF), w.astype(BF),
        (((1,), (1,)), ((), ())), preferred_element_type=F32)


def _softmax_groups(logit):
    e = jnp.exp(logit - jnp.max(logit))
    ri = jax.lax.broadcasted_iota(jnp.int32, (NE, NE), 0) // O
    ci = jax.lax.broadcasted_iota(jnp.int32, (NE, NE), 1) // O
    G8 = (ri == ci).astype(F32)
    gsum = jax.lax.dot_general(G8, e, (((1,), (0,)), ((), ())),
                               preferred_element_type=F32)
    return e / gsum


def _msg_gru_h_body(E_ref, On_ref, Wmn_ref, Wl1_ref, Wl2_ref, Wme_ref,
                    bmn_ref, bl1_ref, bme_ref,
                    Hn_ref, Wih_ref, Whh_ref, bih_ref, bhh_ref,
                    Xh_ref, WhhS_ref, bhhS_ref,
                    out_ref, gh_ref,
                    omsgT_scr, logit_scr, wgt_scr, um_scr,
                    wl1bf_scr, wmebf_scr, msum_scr, gi_scr, hn_scr, hnbf_scr):
    i = pl.program_id(0)

    for j in range(NS):
        def _s_branch(j=j):
            js = slice(j * QS, (j + 1) * QS)
            gh_ref[:, js] = (_bdot(Xh_ref[...], WhhS_ref[...])
                             + bhhS_ref[:, js])
        pl.when(i == j)(_s_branch)

    for b in range(4):
        def _o_branch(b=b):
            cols = slice(b * QW, (b + 1) * QW)
            om = _bdot(On_ref[...], Wmn_ref[...]) + bmn_ref[:, cols]
            omsgT_scr[:, cols] = jnp.concatenate(
                [om, om, om, om], axis=0).astype(BF)
            if b == 0:
                hnbf_scr[...] = Hn_ref[...].astype(BF)
        pl.when(i == b)(_o_branch)

    for b in range(4):
        def _a_branch(b=b):
            cols = slice(b * QW, (b + 1) * QW)
            wl1bf_scr[b * QW:(b + 1) * QW, :] = Wl1_ref[...].astype(BF)
            A = jnp.maximum(_bdot(E_ref[...], Wl1_ref[...]) + bl1_ref[:, cols],
                            0.0)
            part = jnp.sum(A * Wl2_ref[:, cols], axis=1, keepdims=True)
            if b == 0:
                logit_scr[...] = part
            elif b < 3:
                logit_scr[...] += part
            else:
                wgt_scr[...] = _softmax_groups(logit_scr[...] + part)
        pl.when(i == 4 + b)(_a_branch)

    for b in range(4):
        def _m_branch(b=b):
            cols = slice(b * QW, (b + 1) * QW)
            wmebf_scr[b * QW:(b + 1) * QW, :] = Wme_ref[...].astype(BF)
            Em = _bdot(E_ref[...], Wme_ref[...]) + bme_ref[:, cols]
            um_scr[:, cols] = (wgt_scr[...] * Em).astype(BF)
            if b == 3:
                om = omsgT_scr[...].astype(F32)
                um_scr[:, HALF:D] = (wgt_scr[...] * om).astype(BF)
        pl.when(i == 8 + b)(_m_branch)

    def _c1():
        A2 = jnp.maximum(
            jax.lax.dot_general(um_scr[...], wl1bf_scr[...],
                                (((1,), (1,)), ((), ())),
                                preferred_element_type=F32) + bl1_ref[...],
            0.0)
        logit_scr[...] = jnp.sum(A2 * Wl2_ref[...], axis=1, keepdims=True)
    pl.when(i == 12)(_c1)

    def _c2():
        wgt2 = _softmax_groups(logit_scr[...])
        Em2 = jax.lax.dot_general(um_scr[...], wmebf_scr[...],
                                  (((1,), (1,)), ((), ())),
                                  preferred_element_type=F32) + bme_ref[...]
        UM2 = wgt2 * jnp.concatenate(
            [Em2, omsgT_scr[...].astype(F32)], axis=1)
        msum_scr[...] = (jnp.sum(UM2.reshape(NH, O, D), axis=1)
                         * (1.0 / O)).astype(BF)
    pl.when(i == 13)(_c2)

    def _g_stream():
        k = i - MSG_STEPS
        cdims = (((1,), (1,)), ((), ()))
        gi = jax.lax.dot_general(
            msum_scr[...], Wih_ref[...].astype(BF), cdims,
            preferred_element_type=F32)
        hn = jax.lax.dot_general(
            hnbf_scr[...], Whh_ref[...].astype(BF), cdims,
            preferred_element_type=F32)
        gi_scr[pl.ds(k, 1)] = gi.reshape(1, NH, QG)
        hn_scr[pl.ds(k, 1)] = hn.reshape(1, NH, QG)
    pl.when(i >= MSG_STEPS)(_g_stream)

    for c in range(8):
        cs = slice(c * QG, (c + 1) * QG)

        def _g_out(c=c, cs=cs):
            rs = slice(c * QG, (c + 1) * QG)
            zs = slice(D + c * QG, D + (c + 1) * QG)
            ns = slice(2 * D + c * QG, 2 * D + (c + 1) * QG)
            r = jax.nn.sigmoid(gi_scr[c] + bih_ref[:, rs]
                               + hn_scr[c] + bhh_ref[:, rs])
            z = jax.nn.sigmoid(gi_scr[8 + c] + bih_ref[:, zs]
                               + hn_scr[8 + c] + bhh_ref[:, zs])
            n = jnp.tanh(gi_scr[16 + c] + bih_ref[:, ns]
                         + r * (hn_scr[16 + c] + bhh_ref[:, ns]))
            lH = (1.0 - z) * n + z * Hn_ref[:, cs]
            out_ref[:, cs] = 0.25 * (lH[0:NF] + lH[NF:2 * NF]
                                     + lH[2 * NF:3 * NF] + lH[3 * NF:4 * NF])
        pl.when(i == MSG_STEPS + 16 + c)(_g_out)


def _gru_s_body(All_ref, Xh_ref, Wih_ref, gh_ref, bih_ref,
                out_ref, a_scr, b_scr, hs_scr, wbf_scr):
    i = pl.program_id(0)
    gi = _bdot(All_ref[...], Wih_ref[...])
    for k in range(NQ):
        g = k * QB // D
        ks = slice(k * QB, (k + 1) * QB)
        cs = slice(k * QB % D, k * QB % D + QB)

        def _branch(g=g, cs=cs, ks=ks, k=k):
            wbf_scr[k * QB:(k + 1) * QB, :] = Wih_ref[...].astype(BF)
            g1 = gh_ref[0:NF, ks]
            gi1 = gi + bih_ref[:, ks]
            if g == 0:
                a_scr[:, cs] = jax.nn.sigmoid(gi1 + g1)
            elif g == 1:
                b_scr[:, cs] = jax.nn.sigmoid(gi1 + g1)
            else:
                n1 = jnp.tanh(gi1 + a_scr[:, cs] * g1)
                z1 = b_scr[:, cs]
                hs_scr[:, cs] = (1.0 - z1) * n1 + z1 * Xh_ref[0:NF, cs]
        pl.when(i == k)(_branch)

    def _final():
        hs = hs_scr[...].astype(BF)
        gi2 = jax.lax.dot_general(hs, wbf_scr[...], (((1,), (1,)), ((), ())),
                                  preferred_element_type=F32)
        gi2 = gi2 + bih_ref[...]
        hn2 = gh_ref[NF:2 * NF, :]
        r2 = jax.nn.sigmoid(gi2[:, 0:D] + hn2[:, 0:D])
        z2 = jax.nn.sigmoid(gi2[:, D:2 * D] + hn2[:, D:2 * D])
        n2 = jnp.tanh(gi2[:, 2 * D:] + r2 * hn2[:, 2 * D:])
        out_ref[...] = (1.0 - z2) * n2 + z2 * Xh_ref[NF:2 * NF, :]
    pl.when(i == NQ)(_final)


_PARAMS = pltpu.CompilerParams(dimension_semantics=("arbitrary",))


@jax.jit
def kernel(S_node_C4, final_S_node, H_nodes, O_nodes, H_O_edges,
           W_msg_node, b_msg_node, W_msg_edge, b_msg_edge,
           W_l1, b_l1, W_l2, b_l2,
           Wih_H, Whh_H, bih_H, bhh_H,
           Wih_S, Whh_S, bih_S, bhh_S):
    E0 = (H_O_edges.reshape(B, FM, H, O, D)
          .transpose(2, 0, 1, 3, 4).reshape(NE, D))
    On = O_nodes.reshape(NF * O, D)
    Hn = H_nodes.transpose(2, 0, 1, 3).reshape(NH, D)
    sC4 = S_node_C4.reshape(NF, D)
    Sf = final_S_node.transpose(0, 2, 1).reshape(NF, D)
    Xh = jnp.concatenate([sC4, Sf], axis=0)

    full = lambda shape: pl.BlockSpec(shape, lambda i: tuple(0 for _ in shape))
    w_spec = lambda s, n, d: pl.BlockSpec(
        (s, D), lambda i, n=n, d=d: (jnp.clip(i - d, 0, n - 1), 0))

    All, gh = pl.pallas_call(
        _msg_gru_h_body,
        grid=(MSG_STEPS + NGH,),
        in_specs=[full((NE, D)), full((NF * O, D)),
                  w_spec(QW, 4, 0), w_spec(QW, 4, 4), full((1, HALF)),
                  w_spec(QW, 4, 8),
                  full((1, HALF)), full((1, HALF)), full((1, HALF)),
                  full((NH, D)),
                  w_spec(QG, NGH, MSG_STEPS), w_spec(QG, NGH, MSG_STEPS),
                  full((1, G3)), full((1, G3)),
                  full((2 * NF, D)), w_spec(QS, NS, 0), full((1, G3))],
        out_specs=[full((NF, D)), full((2 * NF, G3))],
        out_shape=[jax.ShapeDtypeStruct((NF, D), F32),
                   jax.ShapeDtypeStruct((2 * NF, G3), F32)],
        scratch_shapes=[pltpu.VMEM((NE, HALF), BF),
                        pltpu.VMEM((NE, 1), F32),
                        pltpu.VMEM((NE, 1), F32),
                        pltpu.VMEM((NE, D), BF),
                        pltpu.VMEM((HALF, D), BF),
                        pltpu.VMEM((HALF, D), BF),
                        pltpu.VMEM((NH, D), BF),
                        pltpu.VMEM((NGH, NH, QG), F32),
                        pltpu.VMEM((NGH, NH, QG), F32),
                        pltpu.VMEM((NH, D), BF)],
        compiler_params=_PARAMS,
    )(E0, On, W_msg_node, W_l1, W_l2, W_msg_edge,
      b_msg_node.reshape(1, HALF), b_l1.reshape(1, HALF),
      b_msg_edge.reshape(1, HALF),
      Hn, Wih_H, Whh_H, bih_H.reshape(1, G3), bhh_H.reshape(1, G3),
      Xh, Whh_S, bhh_S.reshape(1, G3))

    q_spec = pl.BlockSpec((QB, D), lambda i: (jnp.minimum(i, NQ - 1), 0))
    S_cls = pl.pallas_call(
        _gru_s_body,
        grid=(NQ + 1,),
        in_specs=[full((NF, D)), full((2 * NF, D)), q_spec,
                  full((2 * NF, G3)), full((1, G3))],
        out_specs=full((NF, D)),
        out_shape=jax.ShapeDtypeStruct((NF, D), F32),
        scratch_shapes=[pltpu.VMEM((NF, D), F32), pltpu.VMEM((NF, D), F32),
                        pltpu.VMEM((NF, D), F32),
                        pltpu.VMEM((G3, D), BF)],
        compiler_params=_PARAMS,
    )(All, Xh, Wih_S, gh, bih_S.reshape(1, G3))

    return S_cls.reshape(B, FM, D)


# speedup vs baseline: 1.0719x; 1.0719x over previous
import jax
import jax.numpy as jnp
from jax.experimental import pallas as pl
from jax.experimental.pallas import tpu as pltpu

B, FM, H, O, D = 2, 8, 4, 8, 2048
HALF = D // 2
NF = B * FM
NE = NF * H * O
NH = NF * H
G3 = 3 * D
QG = 256
NGH = G3 // QG
QB = 512
NQ = G3 // QB
BF = jnp.bfloat16
F32 = jnp.float32


def _bdot(x, w):
    return jax.lax.dot_general(
        x.astype(BF), w.astype(BF),
        (((1,), (1,)), ((), ())), preferred_element_type=F32)


def _softmax_groups(logit):
    e = jnp.exp(logit - jnp.max(logit))
    ri = jax.lax.broadcasted_iota(jnp.int32, (NE, NE), 0) // O
    ci = jax.lax.broadcasted_iota(jnp.int32, (NE, NE), 1) // O
    G8 = (ri == ci).astype(F32)
    gsum = jax.lax.dot_general(G8, e, (((1,), (0,)), ((), ())),
                               preferred_element_type=F32)
    return e / gsum


def _msg_gru_h_body(E_ref, On_ref, Wmn_ref, Wl1_ref, Wl2_ref, Wme_ref,
                    bmn_ref, bl1_ref, bme_ref,
                    Hn_ref, Wih_ref, Whh_ref, bih_ref, bhh_ref,
                    out_ref, msum_scr, gi_scr, hn_scr, hnbf_scr):
    i = pl.program_id(0)

    def _msg():
        hnbf_scr[...] = Hn_ref[...].astype(BF)
        Omsg = _bdot(On_ref[...], Wmn_ref[...]) + bmn_ref[...]
        OmsgT = jnp.concatenate([Omsg, Omsg, Omsg, Omsg], axis=0)
        Ecur = E_ref[...]
        UM = Ecur
        for _ in range(2):
            A = jnp.maximum(_bdot(Ecur, Wl1_ref[...]) + bl1_ref[...], 0.0)
            logit = jnp.sum(A * Wl2_ref[...], axis=1, keepdims=True)
            wgt = _softmax_groups(logit)
            Em = _bdot(Ecur, Wme_ref[...]) + bme_ref[...]
            UM = wgt * jnp.concatenate([Em, OmsgT], axis=1)
            Ecur = UM
        msum_scr[...] = (jnp.sum(UM.reshape(NH, O, D), axis=1)
                         * (1.0 / O)).astype(BF)
    pl.when(i == 0)(_msg)

    def _g_stream():
        k = i - 1
        cdims = (((1,), (1,)), ((), ()))
        gi = jax.lax.dot_general(
            msum_scr[...], Wih_ref[...].astype(BF), cdims,
            preferred_element_type=F32)
        hn = jax.lax.dot_general(
            hnbf_scr[...], Whh_ref[...].astype(BF), cdims,
            preferred_element_type=F32)
        gi_scr[pl.ds(k, 1)] = gi.reshape(1, NH, QG)
        hn_scr[pl.ds(k, 1)] = hn.reshape(1, NH, QG)
    pl.when(i >= 1)(_g_stream)

    for c in range(8):
        cs = slice(c * QG, (c + 1) * QG)

        def _g_out(c=c, cs=cs):
            rs = slice(c * QG, (c + 1) * QG)
            zs = slice(D + c * QG, D + (c + 1) * QG)
            ns = slice(2 * D + c * QG, 2 * D + (c + 1) * QG)
            r = jax.nn.sigmoid(gi_scr[c] + bih_ref[:, rs]
                               + hn_scr[c] + bhh_ref[:, rs])
            z = jax.nn.sigmoid(gi_scr[8 + c] + bih_ref[:, zs]
                               + hn_scr[8 + c] + bhh_ref[:, zs])
            n = jnp.tanh(gi_scr[16 + c] + bih_ref[:, ns]
                         + r * (hn_scr[16 + c] + bhh_ref[:, ns]))
            lH = (1.0 - z) * n + z * Hn_ref[:, cs]
            out_ref[:, cs] = 0.25 * (lH[0:NF] + lH[NF:2 * NF]
                                     + lH[2 * NF:3 * NF] + lH[3 * NF:4 * NF])
        pl.when(i == 17 + c)(_g_out)


def _gru_s_body(All_ref, Xh_ref, Wih_ref, Whh_ref, bih_ref, bhh_ref,
                out_ref, a_scr, b_scr, hs_scr, gh2_scr, wbf_scr):
    i = pl.program_id(0)
    gi = _bdot(All_ref[...], Wih_ref[...])
    gh = _bdot(Xh_ref[...], Whh_ref[...])
    for k in range(NQ):
        g = k * QB // D
        ks = slice(k * QB, (k + 1) * QB)
        cs = slice(k * QB % D, k * QB % D + QB)

        def _branch(g=g, cs=cs, ks=ks, k=k):
            wbf_scr[k * QB:(k + 1) * QB, :] = Wih_ref[...].astype(BF)
            bh = bhh_ref[:, ks]
            gh2_scr[:, ks] = gh[NF:2 * NF] + bh
            g1 = gh[0:NF] + bh
            gi1 = gi + bih_ref[:, ks]
            if g == 0:
                a_scr[:, cs] = jax.nn.sigmoid(gi1 + g1)
            elif g == 1:
                b_scr[:, cs] = jax.nn.sigmoid(gi1 + g1)
            else:
                n1 = jnp.tanh(gi1 + a_scr[:, cs] * g1)
                z1 = b_scr[:, cs]
                hs_scr[:, cs] = (1.0 - z1) * n1 + z1 * Xh_ref[0:NF, cs]
        pl.when(i == k)(_branch)

    def _final():
        hs = hs_scr[...].astype(BF)
        gi2 = jax.lax.dot_general(hs, wbf_scr[...], (((1,), (1,)), ((), ())),
                                  preferred_element_type=F32)
        gi2 = gi2 + bih_ref[...]
        hn2 = gh2_scr[...]
        r2 = jax.nn.sigmoid(gi2[:, 0:D] + hn2[:, 0:D])
        z2 = jax.nn.sigmoid(gi2[:, D:2 * D] + hn2[:, D:2 * D])
        n2 = jnp.tanh(gi2[:, 2 * D:] + r2 * hn2[:, 2 * D:])
        out_ref[...] = (1.0 - z2) * n2 + z2 * Xh_ref[NF:2 * NF, :]
    pl.when(i == NQ)(_final)


_PARAMS = pltpu.CompilerParams(dimension_semantics=("arbitrary",))


@jax.jit
def kernel(S_node_C4, final_S_node, H_nodes, O_nodes, H_O_edges,
           W_msg_node, b_msg_node, W_msg_edge, b_msg_edge,
           W_l1, b_l1, W_l2, b_l2,
           Wih_H, Whh_H, bih_H, bhh_H,
           Wih_S, Whh_S, bih_S, bhh_S):
    E0 = (H_O_edges.reshape(B, FM, H, O, D)
          .transpose(2, 0, 1, 3, 4).reshape(NE, D))
    On = O_nodes.reshape(NF * O, D)
    Hn = H_nodes.transpose(2, 0, 1, 3).reshape(NH, D)
    sC4 = S_node_C4.reshape(NF, D)
    Sf = final_S_node.transpose(0, 2, 1).reshape(NF, D)
    Xh = jnp.concatenate([sC4, Sf], axis=0)

    full = lambda shape: pl.BlockSpec(shape, lambda i: tuple(0 for _ in shape))
    g_spec = pl.BlockSpec((QG, D), lambda i: (jnp.clip(i - 1, 0, NGH - 1), 0))

    All = pl.pallas_call(
        _msg_gru_h_body,
        grid=(1 + NGH,),
        in_specs=[full((NE, D)), full((NF * O, D)),
                  full((HALF, D)), full((HALF, D)), full((1, HALF)),
                  full((HALF, D)),
                  full((1, HALF)), full((1, HALF)), full((1, HALF)),
                  full((NH, D)), g_spec, g_spec,
                  full((1, G3)), full((1, G3))],
        out_specs=full((NF, D)),
        out_shape=jax.ShapeDtypeStruct((NF, D), F32),
        scratch_shapes=[pltpu.VMEM((NH, D), BF),
                        pltpu.VMEM((NGH, NH, QG), F32),
                        pltpu.VMEM((NGH, NH, QG), F32),
                        pltpu.VMEM((NH, D), BF)],
        compiler_params=_PARAMS,
    )(E0, On, W_msg_node, W_l1, W_l2, W_msg_edge,
      b_msg_node.reshape(1, HALF), b_l1.reshape(1, HALF),
      b_msg_edge.reshape(1, HALF),
      Hn, Wih_H, Whh_H, bih_H.reshape(1, G3), bhh_H.reshape(1, G3))

    q_spec = pl.BlockSpec((QB, D), lambda i: (jnp.minimum(i, NQ - 1), 0))
    S_cls = pl.pallas_call(
        _gru_s_body,
        grid=(NQ + 1,),
        in_specs=[full((NF, D)), full((2 * NF, D)), q_spec, q_spec,
                  full((1, G3)), full((1, G3))],
        out_specs=full((NF, D)),
        out_shape=jax.ShapeDtypeStruct((NF, D), F32),
        scratch_shapes=[pltpu.VMEM((NF, D), F32), pltpu.VMEM((NF, D), F32),
                        pltpu.VMEM((NF, D), F32), pltpu.VMEM((NF, G3), F32),
                        pltpu.VMEM((G3, D), BF)],
        compiler_params=_PARAMS,
    )(All, Xh, Wih_S, Whh_S, bih_S.reshape(1, G3), bhh_S.reshape(1, G3))

    return S_cls.reshape(B, FM, D)


# speedup vs baseline: 1.1359x; 1.0596x over previous
import jax
import jax.numpy as jnp
from jax.experimental import pallas as pl
from jax.experimental.pallas import tpu as pltpu

B, FM, H, O, D = 2, 8, 4, 8, 2048
HALF = D // 2
NF = B * FM
NE = NF * H * O
NH = NF * H
G3 = 3 * D
QG = 512
NGH = G3 // QG
QB = 512
NQ = G3 // QB
BF = jnp.bfloat16
F32 = jnp.float32


def _bdot(x, w):
    return jax.lax.dot_general(
        x.astype(BF), w.astype(BF),
        (((1,), (1,)), ((), ())), preferred_element_type=F32)


def _softmax_groups(logit):
    e = jnp.exp(logit - jnp.max(logit))
    ri = jax.lax.broadcasted_iota(jnp.int32, (NE, NE), 0) // O
    ci = jax.lax.broadcasted_iota(jnp.int32, (NE, NE), 1) // O
    G8 = (ri == ci).astype(F32)
    gsum = jax.lax.dot_general(G8, e, (((1,), (0,)), ((), ())),
                               preferred_element_type=F32)
    return e / gsum


def _msg_gru_h_body(E_ref, On_ref, Wmn_ref, Wl1_ref, Wl2_ref, Wme_ref,
                    bmn_ref, bl1_ref, bme_ref,
                    Hn_ref, Wih_ref, Whh_ref, bih_ref, bhh_ref,
                    out_ref, msum_scr, gi_scr, hn_scr, hnbf_scr):
    i = pl.program_id(0)

    def _msg():
        hnbf_scr[...] = Hn_ref[...].astype(BF)
        Omsg = _bdot(On_ref[...], Wmn_ref[...]) + bmn_ref[...]
        OmsgT = jnp.concatenate([Omsg, Omsg, Omsg, Omsg], axis=0)
        Ecur = E_ref[...]
        UM = Ecur
        for _ in range(2):
            A = jnp.maximum(_bdot(Ecur, Wl1_ref[...]) + bl1_ref[...], 0.0)
            logit = jnp.sum(A * Wl2_ref[...], axis=1, keepdims=True)
            wgt = _softmax_groups(logit)
            Em = _bdot(Ecur, Wme_ref[...]) + bme_ref[...]
            UM = wgt * jnp.concatenate([Em, OmsgT], axis=1)
            Ecur = UM
        msum_scr[...] = (jnp.sum(UM.reshape(NH, O, D), axis=1)
                         * (1.0 / O)).astype(BF)
    pl.when(i == 0)(_msg)

    def _g_stream():
        k = i - 1
        cdims = (((1,), (1,)), ((), ()))
        gi = jax.lax.dot_general(
            msum_scr[...], Wih_ref[...].astype(BF), cdims,
            preferred_element_type=F32)
        hn = jax.lax.dot_general(
            hnbf_scr[...], Whh_ref[...].astype(BF), cdims,
            preferred_element_type=F32)
        gi_scr[pl.ds(k, 1)] = gi.reshape(1, NH, QG).astype(BF)
        hn_scr[pl.ds(k, 1)] = hn.reshape(1, NH, QG).astype(BF)
    pl.when(i >= 1)(_g_stream)

    for c in range(4):
        cs = slice(c * QG, (c + 1) * QG)

        def _g_out(c=c, cs=cs):
            rs = slice(c * QG, (c + 1) * QG)
            zs = slice(D + c * QG, D + (c + 1) * QG)
            ns = slice(2 * D + c * QG, 2 * D + (c + 1) * QG)
            r = jax.nn.sigmoid(gi_scr[c].astype(F32) + bih_ref[:, rs]
                               + hn_scr[c].astype(F32) + bhh_ref[:, rs])
            z = jax.nn.sigmoid(gi_scr[4 + c].astype(F32) + bih_ref[:, zs]
                               + hn_scr[4 + c].astype(F32) + bhh_ref[:, zs])
            n = jnp.tanh(gi_scr[8 + c].astype(F32) + bih_ref[:, ns]
                         + r * (hn_scr[8 + c].astype(F32) + bhh_ref[:, ns]))
            lH = (1.0 - z) * n + z * Hn_ref[:, cs]
            out_ref[:, cs] = 0.25 * (lH[0:NF] + lH[NF:2 * NF]
                                     + lH[2 * NF:3 * NF] + lH[3 * NF:4 * NF])
        pl.when(i == 9 + c)(_g_out)


def _gru_s_body(All_ref, Xh_ref, Wih_ref, Whh_ref, bih_ref, bhh_ref,
                out_ref, a_scr, b_scr, hs_scr, gh2_scr, wbf_scr):
    i = pl.program_id(0)
    gi = _bdot(All_ref[...], Wih_ref[...])
    gh = _bdot(Xh_ref[...], Whh_ref[...])
    for k in range(NQ):
        g = k * QB // D
        ks = slice(k * QB, (k + 1) * QB)
        cs = slice(k * QB % D, k * QB % D + QB)

        def _branch(g=g, cs=cs, ks=ks, k=k):
            wbf_scr[k * QB:(k + 1) * QB, :] = Wih_ref[...].astype(BF)
            bh = bhh_ref[:, ks]
            gh2_scr[:, ks] = gh[NF:2 * NF] + bh
            g1 = gh[0:NF] + bh
            gi1 = gi + bih_ref[:, ks]
            if g == 0:
                a_scr[:, cs] = jax.nn.sigmoid(gi1 + g1)
            elif g == 1:
                b_scr[:, cs] = jax.nn.sigmoid(gi1 + g1)
            else:
                n1 = jnp.tanh(gi1 + a_scr[:, cs] * g1)
                z1 = b_scr[:, cs]
                hs_scr[:, cs] = (1.0 - z1) * n1 + z1 * Xh_ref[0:NF, cs]
        pl.when(i == k)(_branch)

    def _final():
        hs = hs_scr[...].astype(BF)
        gi2 = jax.lax.dot_general(hs, wbf_scr[...], (((1,), (1,)), ((), ())),
                                  preferred_element_type=F32)
        gi2 = gi2 + bih_ref[...]
        hn2 = gh2_scr[...]
        r2 = jax.nn.sigmoid(gi2[:, 0:D] + hn2[:, 0:D])
        z2 = jax.nn.sigmoid(gi2[:, D:2 * D] + hn2[:, D:2 * D])
        n2 = jnp.tanh(gi2[:, 2 * D:] + r2 * hn2[:, 2 * D:])
        out_ref[...] = (1.0 - z2) * n2 + z2 * Xh_ref[NF:2 * NF, :]
    pl.when(i == NQ)(_final)


_PARAMS = pltpu.CompilerParams(dimension_semantics=("arbitrary",))


@jax.jit
def kernel(S_node_C4, final_S_node, H_nodes, O_nodes, H_O_edges,
           W_msg_node, b_msg_node, W_msg_edge, b_msg_edge,
           W_l1, b_l1, W_l2, b_l2,
           Wih_H, Whh_H, bih_H, bhh_H,
           Wih_S, Whh_S, bih_S, bhh_S):
    E0 = (H_O_edges.reshape(B, FM, H, O, D)
          .transpose(2, 0, 1, 3, 4).reshape(NE, D))
    On = O_nodes.reshape(NF * O, D)
    Hn = H_nodes.transpose(2, 0, 1, 3).reshape(NH, D)
    sC4 = S_node_C4.reshape(NF, D)
    Sf = final_S_node.transpose(0, 2, 1).reshape(NF, D)
    Xh = jnp.concatenate([sC4, Sf], axis=0)

    full = lambda shape: pl.BlockSpec(shape, lambda i: tuple(0 for _ in shape))
    g_spec = pl.BlockSpec((QG, D), lambda i: (jnp.clip(i - 1, 0, NGH - 1), 0))

    All = pl.pallas_call(
        _msg_gru_h_body,
        grid=(1 + NGH,),
        in_specs=[full((NE, D)), full((NF * O, D)),
                  full((HALF, D)), full((HALF, D)), full((1, HALF)),
                  full((HALF, D)),
                  full((1, HALF)), full((1, HALF)), full((1, HALF)),
                  full((NH, D)), g_spec, g_spec,
                  full((1, G3)), full((1, G3))],
        out_specs=full((NF, D)),
        out_shape=jax.ShapeDtypeStruct((NF, D), F32),
        scratch_shapes=[pltpu.VMEM((NH, D), BF),
                        pltpu.VMEM((NGH, NH, QG), BF),
                        pltpu.VMEM((NGH, NH, QG), BF),
                        pltpu.VMEM((NH, D), BF)],
        compiler_params=_PARAMS,
    )(E0, On, W_msg_node, W_l1, W_l2, W_msg_edge,
      b_msg_node.reshape(1, HALF), b_l1.reshape(1, HALF),
      b_msg_edge.reshape(1, HALF),
      Hn, Wih_H, Whh_H, bih_H.reshape(1, G3), bhh_H.reshape(1, G3))

    q_spec = pl.BlockSpec((QB, D), lambda i: (jnp.minimum(i, NQ - 1), 0))
    S_cls = pl.pallas_call(
        _gru_s_body,
        grid=(NQ + 1,),
        in_specs=[full((NF, D)), full((2 * NF, D)), q_spec, q_spec,
                  full((1, G3)), full((1, G3))],
        out_specs=full((NF, D)),
        out_shape=jax.ShapeDtypeStruct((NF, D), F32),
        scratch_shapes=[pltpu.VMEM((NF, D), F32), pltpu.VMEM((NF, D), F32),
                        pltpu.VMEM((NF, D), F32), pltpu.VMEM((NF, G3), F32),
                        pltpu.VMEM((G3, D), BF)],
        compiler_params=_PARAMS,
    )(All, Xh, Wih_S, Whh_S, bih_S.reshape(1, G3), bhh_S.reshape(1, G3))

    return S_cls.reshape(B, FM, D)
